# Initial kernel scaffold; baseline (speedup 1.0000x reference)
import jax
import jax.numpy as jnp
from jax.experimental import pallas as pl


def kernel(x, batch, W1, b1, W2, b2, Wc, bc):
    raise NotImplementedError("write your pallas kernel here")


# speedup vs baseline: 4.7736x; 4.7736x over previous
import functools

import jax
import jax.numpy as jnp
from jax import lax
from jax.experimental import pallas as pl
from jax.experimental.pallas import tpu as pltpu
from jax.experimental.pallas import tpu_sc as plsc

N = 100000
D = 128
G = 128
H = 64

B = 2000
NB = N // B

NEG = -1e30


def _stats_body(x_ref, b_ref, w1_ref, b1_ref, w2_ref, b2_ref, wc_ref, bc_ref,
                out_ref, m_ref, d_ref, s_ref):
    i = pl.program_id(0)

    @pl.when(i == 0)
    def _():
        m_ref[...] = jnp.full((G, 1), NEG, jnp.float32)
        d_ref[...] = jnp.zeros((G, 1), jnp.float32)
        s_ref[...] = jnp.zeros((G, D), jnp.float32)

    xb = x_ref[...]
    bb = b_ref[0]

    h = jnp.tanh(jnp.dot(xb, w1_ref[...], preferred_element_type=jnp.float32)
                 + b1_ref[...])
    s_row = lax.dot_general(w2_ref[...], h, (((0,), (1,)), ((), ())),
                            preferred_element_type=jnp.float32) + b2_ref[...]

    gi = lax.broadcasted_iota(jnp.int32, (G, B), 0)
    oh = gi == bb

    masked = jnp.where(oh, s_row, NEG)
    bmax = jnp.max(masked, axis=1, keepdims=True)
    m_old = m_ref[...]
    m_new = jnp.maximum(m_old, bmax)
    scale = jnp.exp(m_old - m_new)
    e = jnp.exp(jnp.where(oh, s_row - m_new, NEG))
    bd = jnp.sum(e, axis=1, keepdims=True)
    d_ref[...] = d_ref[...] * scale + bd
    sb = lax.dot_general(e, xb, (((1,), (0,)), ((), ())),
                         preferred_element_type=jnp.float32)
    s_ref[...] = s_ref[...] * scale + sb
    m_ref[...] = m_new

    @pl.when(i == NB - 1)
    def _():
        dd = d_ref[...]
        r = 1.0 / jnp.where(dd > 0, dd, 1.0)
        ge = s_ref[...] * r
        out_ref[...] = jnp.dot(ge, wc_ref[...],
                               preferred_element_type=jnp.float32) + bc_ref[...]


_context_call = pl.pallas_call(
    _stats_body,
    grid=(NB,),
    in_specs=[
        pl.BlockSpec((B, D), lambda i: (i, 0)),
        pl.BlockSpec((1, 1, B), lambda i: (i, 0, 0)),
        pl.BlockSpec((D, H), lambda i: (0, 0)),
        pl.BlockSpec((1, H), lambda i: (0, 0)),
        pl.BlockSpec((H, 1), lambda i: (0, 0)),
        pl.BlockSpec((1, 1), lambda i: (0, 0)),
        pl.BlockSpec((D, D), lambda i: (0, 0)),
        pl.BlockSpec((1, D), lambda i: (0, 0)),
    ],
    out_specs=pl.BlockSpec((G, D), lambda i: (0, 0)),
    out_shape=jax.ShapeDtypeStruct((G, D), jnp.float32),
    scratch_shapes=[
        pltpu.VMEM((G, 1), jnp.float32),
        pltpu.VMEM((G, 1), jnp.float32),
        pltpu.VMEM((G, D), jnp.float32),
    ],
)


NC, NS = 2, 16
NW = NC * NS
C = 80
NCHUNK = N // C
_CPW = -(-NCHUNK // NW)
NCHUNK_PAD = NW * _CPW

@functools.cache
def _gather_ctx_call():
    mesh = plsc.VectorSubcoreMesh(core_axis_name="c", subcore_axis_name="s",
                                  num_cores=NC, num_subcores=NS)

    @functools.partial(
        pl.kernel,
        out_type=jax.ShapeDtypeStruct((N, D), jnp.float32),
        mesh=mesh,
        scratch_types=[
            pltpu.VMEM((_CPW, C), jnp.int32),
            pltpu.VMEM((C, D), jnp.float32),
            pltpu.SemaphoreType.DMA,
        ],
    )
    def _gather_ctx(ctx_hbm, idx_hbm, out_hbm, idx_v, rows_v, sem):
        wid = lax.axis_index("s") * NC + lax.axis_index("c")
        start = wid * _CPW
        nmine = jnp.clip(NCHUNK - start, 0, _CPW)

        pltpu.sync_copy(idx_hbm.at[pl.ds(start, _CPW)], idx_v)

        def body(j):
            @pl.when(j < nmine)
            def _():
                base = (start + j) * C
                pltpu.async_copy(ctx_hbm.at[idx_v.at[j]], rows_v, sem).wait()
                pltpu.sync_copy(rows_v, out_hbm.at[pl.ds(base, C)])

        pl.loop(0, _CPW)(body)

    return _gather_ctx


def kernel(x, batch, W1, b1, W2, b2, Wc, bc):
    batch_blk = batch.reshape(NB, 1, B)
    context = _context_call(x, batch_blk, W1, b1.reshape(1, H),
                            W2, b2.reshape(1, 1), Wc, bc.reshape(1, D))
    idx2d = jnp.zeros((NCHUNK_PAD, C), jnp.int32).at[:NCHUNK].set(
        batch.reshape(NCHUNK, C))
    return _gather_ctx_call()(context, idx2d)


# speedup vs baseline: 5.1482x; 1.0785x over previous
import functools

import jax
import jax.numpy as jnp
from jax import lax
from jax.experimental import pallas as pl
from jax.experimental.pallas import tpu as pltpu
from jax.experimental.pallas import tpu_sc as plsc

N = 100000
D = 128
G = 128
H = 64

B = 2000
NB = N // B

NEG = -1e30


def _stats_body(x_ref, b_ref, w1_ref, b1_ref, w2_ref, b2_ref, wc_ref, bc_ref,
                out_ref, m_ref, d_ref, s_ref):
    i = pl.program_id(0)

    @pl.when(i == 0)
    def _():
        m_ref[...] = jnp.full((G, 1), NEG, jnp.float32)
        d_ref[...] = jnp.zeros((G, 1), jnp.float32)
        s_ref[...] = jnp.zeros((G, D), jnp.float32)

    xb = x_ref[...]
    bb = b_ref[0]

    h = jnp.tanh(jnp.dot(xb, w1_ref[...], preferred_element_type=jnp.float32)
                 + b1_ref[...])
    s_row = lax.dot_general(w2_ref[...], h, (((0,), (1,)), ((), ())),
                            preferred_element_type=jnp.float32) + b2_ref[...]

    gi = lax.broadcasted_iota(jnp.int32, (G, B), 0)
    oh = gi == bb

    masked = jnp.where(oh, s_row, NEG)
    bmax = jnp.max(masked, axis=1, keepdims=True)
    m_old = m_ref[...]
    m_new = jnp.maximum(m_old, bmax)
    scale = jnp.exp(m_old - m_new)
    e = jnp.exp(jnp.where(oh, s_row - m_new, NEG))
    bd = jnp.sum(e, axis=1, keepdims=True)
    d_ref[...] = d_ref[...] * scale + bd
    sb = lax.dot_general(e, xb, (((1,), (0,)), ((), ())),
                         preferred_element_type=jnp.float32)
    s_ref[...] = s_ref[...] * scale + sb
    m_ref[...] = m_new

    @pl.when(i == NB - 1)
    def _():
        dd = d_ref[...]
        r = 1.0 / jnp.where(dd > 0, dd, 1.0)
        ge = s_ref[...] * r
        out_ref[...] = jnp.dot(ge, wc_ref[...],
                               preferred_element_type=jnp.float32) + bc_ref[...]


_context_call = pl.pallas_call(
    _stats_body,
    grid=(NB,),
    in_specs=[
        pl.BlockSpec((B, D), lambda i: (i, 0)),
        pl.BlockSpec((1, 1, B), lambda i: (i, 0, 0)),
        pl.BlockSpec((D, H), lambda i: (0, 0)),
        pl.BlockSpec((1, H), lambda i: (0, 0)),
        pl.BlockSpec((H, 1), lambda i: (0, 0)),
        pl.BlockSpec((1, 1), lambda i: (0, 0)),
        pl.BlockSpec((D, D), lambda i: (0, 0)),
        pl.BlockSpec((1, D), lambda i: (0, 0)),
    ],
    out_specs=pl.BlockSpec((G, D), lambda i: (0, 0)),
    out_shape=jax.ShapeDtypeStruct((G, D), jnp.float32),
    scratch_shapes=[
        pltpu.VMEM((G, 1), jnp.float32),
        pltpu.VMEM((G, 1), jnp.float32),
        pltpu.VMEM((G, D), jnp.float32),
    ],
)


NC, NS = 2, 16
NW = NC * NS
C = 80
NCHUNK = N // C
_CPW = -(-NCHUNK // NW)
NCHUNK_PAD = NW * _CPW
_KF = 5
_SUPER = _CPW // _KF
_SR = _KF * C

@functools.cache
def _gather_ctx_call():
    mesh = plsc.VectorSubcoreMesh(core_axis_name="c", subcore_axis_name="s",
                                  num_cores=NC, num_subcores=NS)

    @functools.partial(
        pl.kernel,
        out_type=jax.ShapeDtypeStruct((N, D), jnp.float32),
        mesh=mesh,
        scratch_types=[
            pltpu.VMEM((_CPW, C), jnp.int32),
            pltpu.VMEM((_SR, D), jnp.float32),
            pltpu.SemaphoreType.DMA,
        ],
    )
    def _gather_ctx(ctx_hbm, idx_hbm, out_hbm, idx_v, rows_v, sem):
        wid = lax.axis_index("s") * NC + lax.axis_index("c")
        start = wid * _CPW
        nsuper = jnp.clip(NCHUNK - start, 0, _CPW) // _KF

        pltpu.sync_copy(idx_hbm.at[pl.ds(start, _CPW)], idx_v)

        def body(s):
            @pl.when(s < nsuper)
            def _():
                copies = []
                for k in range(_KF):
                    copies.append(pltpu.async_copy(
                        ctx_hbm.at[idx_v.at[s * _KF + k]],
                        rows_v.at[pl.ds(k * C, C)], sem))
                for cp in copies:
                    cp.wait()
                base = (start + s * _KF) * C
                pltpu.sync_copy(rows_v, out_hbm.at[pl.ds(base, _SR)])

        pl.loop(0, _SUPER)(body)

    return _gather_ctx


def kernel(x, batch, W1, b1, W2, b2, Wc, bc):
    batch_blk = batch.reshape(NB, 1, B)
    context = _context_call(x, batch_blk, W1, b1.reshape(1, H),
                            W2, b2.reshape(1, 1), Wc, bc.reshape(1, D))
    idx2d = jnp.zeros((NCHUNK_PAD, C), jnp.int32).at[:NCHUNK].set(
        batch.reshape(NCHUNK, C))
    return _gather_ctx_call()(context, idx2d)


# speedup vs baseline: 14.0272x; 2.7247x over previous
import functools

import jax
import jax.numpy as jnp
from jax import lax
from jax.experimental import pallas as pl
from jax.experimental.pallas import tpu as pltpu
from jax.experimental.pallas import tpu_sc as plsc

N = 100000
D = 128
G = 128
H = 64

B = 2000
NB = N // B

NEG = -1e30


def _stats_body(x_ref, b_ref, w1_ref, b1_ref, w2_ref, b2_ref, wc_ref, bc_ref,
                out_ref, m_ref, d_ref, s_ref):
    i = pl.program_id(0)

    @pl.when(i == 0)
    def _():
        m_ref[...] = jnp.full((G, 1), NEG, jnp.float32)
        d_ref[...] = jnp.zeros((G, 1), jnp.float32)
        s_ref[...] = jnp.zeros((G, D), jnp.float32)

    xb = x_ref[...]
    bb = b_ref[0]

    h = jnp.tanh(jnp.dot(xb, w1_ref[...], preferred_element_type=jnp.float32)
                 + b1_ref[...])
    s_row = lax.dot_general(w2_ref[...], h, (((0,), (1,)), ((), ())),
                            preferred_element_type=jnp.float32) + b2_ref[...]

    gi = lax.broadcasted_iota(jnp.int32, (G, B), 0)
    oh = gi == bb

    masked = jnp.where(oh, s_row, NEG)
    bmax = jnp.max(masked, axis=1, keepdims=True)
    m_old = m_ref[...]
    m_new = jnp.maximum(m_old, bmax)
    scale = jnp.exp(m_old - m_new)
    e = jnp.exp(jnp.where(oh, s_row - m_new, NEG))
    bd = jnp.sum(e, axis=1, keepdims=True)
    d_ref[...] = d_ref[...] * scale + bd
    sb = lax.dot_general(e, xb, (((1,), (0,)), ((), ())),
                         preferred_element_type=jnp.float32)
    s_ref[...] = s_ref[...] * scale + sb
    m_ref[...] = m_new

    @pl.when(i == NB - 1)
    def _():
        dd = d_ref[...]
        r = 1.0 / jnp.where(dd > 0, dd, 1.0)
        ge = s_ref[...] * r
        out_ref[...] = jnp.dot(ge, wc_ref[...],
                               preferred_element_type=jnp.float32) + bc_ref[...]


_context_call = pl.pallas_call(
    _stats_body,
    grid=(NB,),
    in_specs=[
        pl.BlockSpec((B, D), lambda i: (i, 0)),
        pl.BlockSpec((1, 1, B), lambda i: (i, 0, 0)),
        pl.BlockSpec((D, H), lambda i: (0, 0)),
        pl.BlockSpec((1, H), lambda i: (0, 0)),
        pl.BlockSpec((H, 1), lambda i: (0, 0)),
        pl.BlockSpec((1, 1), lambda i: (0, 0)),
        pl.BlockSpec((D, D), lambda i: (0, 0)),
        pl.BlockSpec((1, D), lambda i: (0, 0)),
    ],
    out_specs=pl.BlockSpec((G, D), lambda i: (0, 0)),
    out_shape=jax.ShapeDtypeStruct((G, D), jnp.float32),
    scratch_shapes=[
        pltpu.VMEM((G, 1), jnp.float32),
        pltpu.VMEM((G, 1), jnp.float32),
        pltpu.VMEM((G, D), jnp.float32),
    ],
)


NC, NS = 2, 16
NW = NC * NS
C = 80
NCHUNK = N // C
_CPW = -(-NCHUNK // NW)
NCHUNK_PAD = NW * _CPW
_KF = 5
_SUPER = _CPW // _KF
_SR = _KF * C

@functools.cache
def _gather_ctx_call():
    mesh = plsc.VectorSubcoreMesh(core_axis_name="c", subcore_axis_name="s",
                                  num_cores=NC, num_subcores=NS)

    @functools.partial(
        pl.kernel,
        out_type=jax.ShapeDtypeStruct((N, D), jnp.float32),
        mesh=mesh,
        scratch_types=[
            pltpu.VMEM((_CPW, C), jnp.int32),
            pltpu.VMEM((_SR, D), jnp.float32),
            pltpu.VMEM_SHARED((G, D), jnp.float32),
            pltpu.SemaphoreType.DMA,
        ],
    )
    def _gather_ctx(ctx_hbm, idx_hbm, out_hbm, idx_v, rows_v, tbl_sh, sem):
        wid = lax.axis_index("s") * NC + lax.axis_index("c")
        start = wid * _CPW
        nsuper = jnp.clip(NCHUNK - start, 0, _CPW) // _KF

        @pl.when(lax.axis_index("s") == 0)
        def _():
            pltpu.sync_copy(ctx_hbm, tbl_sh)

        pltpu.sync_copy(idx_hbm.at[pl.ds(start, _CPW)], idx_v)
        plsc.subcore_barrier()

        def body(s):
            @pl.when(s < nsuper)
            def _():
                copies = []
                for k in range(_KF):
                    copies.append(pltpu.async_copy(
                        tbl_sh.at[idx_v.at[s * _KF + k]],
                        rows_v.at[pl.ds(k * C, C)], sem))
                for cp in copies:
                    cp.wait()
                base = (start + s * _KF) * C
                pltpu.sync_copy(rows_v, out_hbm.at[pl.ds(base, _SR)])

        pl.loop(0, _SUPER)(body)

    return _gather_ctx


def kernel(x, batch, W1, b1, W2, b2, Wc, bc):
    batch_blk = batch.reshape(NB, 1, B)
    context = _context_call(x, batch_blk, W1, b1.reshape(1, H),
                            W2, b2.reshape(1, 1), Wc, bc.reshape(1, D))
    idx2d = jnp.zeros((NCHUNK_PAD, C), jnp.int32).at[:NCHUNK].set(
        batch.reshape(NCHUNK, C))
    return _gather_ctx_call()(context, idx2d)


# speedup vs baseline: 14.4856x; 1.0327x over previous
import functools

import jax
import jax.numpy as jnp
from jax import lax
from jax.experimental import pallas as pl
from jax.experimental.pallas import tpu as pltpu
from jax.experimental.pallas import tpu_sc as plsc

N = 100000
D = 128
G = 128
H = 64

B = 2000
NB = N // B

NEG = -1e30


def _stats_body(x_ref, b_ref, w1_ref, b1_ref, w2_ref, b2_ref, wc_ref, bc_ref,
                out_ref, m_ref, d_ref, s_ref):
    i = pl.program_id(0)

    @pl.when(i == 0)
    def _():
        m_ref[...] = jnp.full((G, 1), NEG, jnp.float32)
        d_ref[...] = jnp.zeros((G, 1), jnp.float32)
        s_ref[...] = jnp.zeros((G, D), jnp.float32)

    xb = x_ref[...]
    bb = b_ref[0]

    h = jnp.tanh(jnp.dot(xb, w1_ref[...], preferred_element_type=jnp.float32)
                 + b1_ref[...])
    s_row = lax.dot_general(w2_ref[...], h, (((0,), (1,)), ((), ())),
                            preferred_element_type=jnp.float32) + b2_ref[...]

    lo = bb[0, 0]
    hi = bb[0, B - 1]
    GW = 32
    for gc in range(G // GW):
        g0 = gc * GW

        @pl.when(jnp.logical_and(lo < g0 + GW, hi >= g0))
        def _(g0=g0):
            gi = lax.broadcasted_iota(jnp.int32, (GW, B), 0) + g0
            oh = gi == bb
            masked = jnp.where(oh, s_row, NEG)
            bmax = jnp.max(masked, axis=1, keepdims=True)
            m_old = m_ref[g0:g0 + GW, :]
            m_new = jnp.maximum(m_old, bmax)
            scale = jnp.exp(m_old - m_new)
            e = jnp.exp(jnp.where(oh, s_row - m_new, NEG))
            bd = jnp.sum(e, axis=1, keepdims=True)
            d_ref[g0:g0 + GW, :] = d_ref[g0:g0 + GW, :] * scale + bd
            sb = lax.dot_general(e, xb, (((1,), (0,)), ((), ())),
                                 preferred_element_type=jnp.float32)
            s_ref[g0:g0 + GW, :] = s_ref[g0:g0 + GW, :] * scale + sb
            m_ref[g0:g0 + GW, :] = m_new

    @pl.when(i == NB - 1)
    def _():
        dd = d_ref[...]
        r = 1.0 / jnp.where(dd > 0, dd, 1.0)
        ge = s_ref[...] * r
        out_ref[...] = jnp.dot(ge, wc_ref[...],
                               preferred_element_type=jnp.float32) + bc_ref[...]


_context_call = pl.pallas_call(
    _stats_body,
    grid=(NB,),
    in_specs=[
        pl.BlockSpec((B, D), lambda i: (i, 0)),
        pl.BlockSpec((1, 1, B), lambda i: (i, 0, 0)),
        pl.BlockSpec((D, H), lambda i: (0, 0)),
        pl.BlockSpec((1, H), lambda i: (0, 0)),
        pl.BlockSpec((H, 1), lambda i: (0, 0)),
        pl.BlockSpec((1, 1), lambda i: (0, 0)),
        pl.BlockSpec((D, D), lambda i: (0, 0)),
        pl.BlockSpec((1, D), lambda i: (0, 0)),
    ],
    out_specs=pl.BlockSpec((G, D), lambda i: (0, 0)),
    out_shape=jax.ShapeDtypeStruct((G, D), jnp.float32),
    scratch_shapes=[
        pltpu.VMEM((G, 1), jnp.float32),
        pltpu.VMEM((G, 1), jnp.float32),
        pltpu.VMEM((G, D), jnp.float32),
    ],
)


NC, NS = 2, 16
NW = NC * NS
C = 80
NCHUNK = N // C
_CPW = -(-NCHUNK // NW)
NCHUNK_PAD = NW * _CPW
_KF = 5
_SUPER = _CPW // _KF
_SR = _KF * C

@functools.cache
def _gather_ctx_call():
    mesh = plsc.VectorSubcoreMesh(core_axis_name="c", subcore_axis_name="s",
                                  num_cores=NC, num_subcores=NS)

    @functools.partial(
        pl.kernel,
        out_type=jax.ShapeDtypeStruct((N, D), jnp.float32),
        mesh=mesh,
        scratch_types=[
            pltpu.VMEM((_CPW, C), jnp.int32),
            pltpu.VMEM((_SR, D), jnp.float32),
            pltpu.VMEM_SHARED((G, D), jnp.float32),
            pltpu.SemaphoreType.DMA,
        ],
    )
    def _gather_ctx(ctx_hbm, idx_hbm, out_hbm, idx_v, rows_v, tbl_sh, sem):
        wid = lax.axis_index("s") * NC + lax.axis_index("c")
        start = wid * _CPW
        nsuper = jnp.clip(NCHUNK - start, 0, _CPW) // _KF

        @pl.when(lax.axis_index("s") == 0)
        def _():
            pltpu.sync_copy(ctx_hbm, tbl_sh)

        pltpu.sync_copy(idx_hbm.at[pl.ds(start, _CPW)], idx_v)
        plsc.subcore_barrier()

        def body(s):
            @pl.when(s < nsuper)
            def _():
                copies = []
                for k in range(_KF):
                    copies.append(pltpu.async_copy(
                        tbl_sh.at[idx_v.at[s * _KF + k]],
                        rows_v.at[pl.ds(k * C, C)], sem))
                for cp in copies:
                    cp.wait()
                base = (start + s * _KF) * C
                pltpu.sync_copy(rows_v, out_hbm.at[pl.ds(base, _SR)])

        pl.loop(0, _SUPER)(body)

    return _gather_ctx


def kernel(x, batch, W1, b1, W2, b2, Wc, bc):
    batch_blk = batch.reshape(NB, 1, B)
    context = _context_call(x, batch_blk, W1, b1.reshape(1, H),
                            W2, b2.reshape(1, 1), Wc, bc.reshape(1, D))
    idx2d = jnp.zeros((NCHUNK_PAD, C), jnp.int32).at[:NCHUNK].set(
        batch.reshape(NCHUNK, C))
    return _gather_ctx_call()(context, idx2d)


# speedup vs baseline: 17.2064x; 1.1878x over previous
import functools

import jax
import jax.numpy as jnp
from jax import lax
from jax.experimental import pallas as pl
from jax.experimental.pallas import tpu as pltpu
from jax.experimental.pallas import tpu_sc as plsc

N = 100000
D = 128
G = 128
H = 64

B = 4000
NB = N // B

NEG = -1e30


def _stats_body(x_ref, b_ref, w1_ref, b1_ref, w2_ref, b2_ref, wc_ref, bc_ref,
                out_ref, m_ref, d_ref, s_ref):
    i = pl.program_id(0)

    @pl.when(i == 0)
    def _():
        m_ref[...] = jnp.full((G, 1), NEG, jnp.float32)
        d_ref[...] = jnp.zeros((G, 1), jnp.float32)
        s_ref[...] = jnp.zeros((G, D), jnp.float32)

    xb = x_ref[...]
    xb_bf = xb.astype(jnp.bfloat16)
    bb = b_ref[0]

    h = jnp.tanh(jnp.dot(xb_bf, w1_ref[...], preferred_element_type=jnp.float32)
                 + b1_ref[...])
    s_row = lax.dot_general(w2_ref[...], h, (((0,), (1,)), ((), ())),
                            preferred_element_type=jnp.float32) + b2_ref[...]

    lo = bb[0, 0]
    hi = bb[0, B - 1]
    GW = 32
    for gc in range(G // GW):
        g0 = gc * GW

        @pl.when(jnp.logical_and(lo < g0 + GW, hi >= g0))
        def _(g0=g0):
            gi = lax.broadcasted_iota(jnp.int32, (GW, B), 0) + g0
            oh = gi == bb
            masked = jnp.where(oh, s_row, NEG)
            bmax = jnp.max(masked, axis=1, keepdims=True)
            m_old = m_ref[g0:g0 + GW, :]
            m_new = jnp.maximum(m_old, bmax)
            scale = jnp.exp(m_old - m_new)
            e = jnp.exp(jnp.where(oh, s_row - m_new, NEG))
            bd = jnp.sum(e, axis=1, keepdims=True)
            d_ref[g0:g0 + GW, :] = d_ref[g0:g0 + GW, :] * scale + bd
            sb = lax.dot_general(e.astype(jnp.bfloat16), xb_bf,
                                 (((1,), (0,)), ((), ())),
                                 preferred_element_type=jnp.float32)
            s_ref[g0:g0 + GW, :] = s_ref[g0:g0 + GW, :] * scale + sb
            m_ref[g0:g0 + GW, :] = m_new

    @pl.when(i == NB - 1)
    def _():
        dd = d_ref[...]
        r = 1.0 / jnp.where(dd > 0, dd, 1.0)
        ge = s_ref[...] * r
        out_ref[...] = jnp.dot(ge, wc_ref[...],
                               preferred_element_type=jnp.float32) + bc_ref[...]


_context_call = pl.pallas_call(
    _stats_body,
    grid=(NB,),
    in_specs=[
        pl.BlockSpec((B, D), lambda i: (i, 0)),
        pl.BlockSpec((1, 1, B), lambda i: (i, 0, 0)),
        pl.BlockSpec((D, H), lambda i: (0, 0)),
        pl.BlockSpec((1, H), lambda i: (0, 0)),
        pl.BlockSpec((H, 1), lambda i: (0, 0)),
        pl.BlockSpec((1, 1), lambda i: (0, 0)),
        pl.BlockSpec((D, D), lambda i: (0, 0)),
        pl.BlockSpec((1, D), lambda i: (0, 0)),
    ],
    out_specs=pl.BlockSpec((G, D), lambda i: (0, 0)),
    out_shape=jax.ShapeDtypeStruct((G, D), jnp.float32),
    scratch_shapes=[
        pltpu.VMEM((G, 1), jnp.float32),
        pltpu.VMEM((G, 1), jnp.float32),
        pltpu.VMEM((G, D), jnp.float32),
    ],
)


NC, NS = 2, 16
NW = NC * NS
C = 80
NCHUNK = N // C
_CPW = -(-NCHUNK // NW)
NCHUNK_PAD = NW * _CPW
_KF = 5
_SUPER = _CPW // _KF
_SR = _KF * C

@functools.cache
def _gather_ctx_call():
    mesh = plsc.VectorSubcoreMesh(core_axis_name="c", subcore_axis_name="s",
                                  num_cores=NC, num_subcores=NS)

    @functools.partial(
        pl.kernel,
        out_type=jax.ShapeDtypeStruct((N, D), jnp.float32),
        mesh=mesh,
        scratch_types=[
            pltpu.VMEM((_CPW, C), jnp.int32),
            pltpu.VMEM((_SR, D), jnp.float32),
            pltpu.VMEM_SHARED((G, D), jnp.float32),
            pltpu.SemaphoreType.DMA,
        ],
    )
    def _gather_ctx(ctx_hbm, idx_hbm, out_hbm, idx_v, rows_v, tbl_sh, sem):
        wid = lax.axis_index("s") * NC + lax.axis_index("c")
        start = wid * _CPW
        nsuper = jnp.clip(NCHUNK - start, 0, _CPW) // _KF

        @pl.when(lax.axis_index("s") == 0)
        def _():
            pltpu.sync_copy(ctx_hbm, tbl_sh)

        pltpu.sync_copy(idx_hbm.at[pl.ds(start, _CPW)], idx_v)
        plsc.subcore_barrier()

        def body(s):
            @pl.when(s < nsuper)
            def _():
                copies = []
                for k in range(_KF):
                    copies.append(pltpu.async_copy(
                        tbl_sh.at[idx_v.at[s * _KF + k]],
                        rows_v.at[pl.ds(k * C, C)], sem))
                for cp in copies:
                    cp.wait()
                base = (start + s * _KF) * C
                pltpu.sync_copy(rows_v, out_hbm.at[pl.ds(base, _SR)])

        pl.loop(0, _SUPER)(body)

    return _gather_ctx


def kernel(x, batch, W1, b1, W2, b2, Wc, bc):
    batch_blk = batch.reshape(NB, 1, B)
    context = _context_call(x, batch_blk, W1.astype(jnp.bfloat16),
                            b1.reshape(1, H),
                            W2, b2.reshape(1, 1), Wc, bc.reshape(1, D))
    idx2d = jnp.zeros((NCHUNK_PAD, C), jnp.int32).at[:NCHUNK].set(
        batch.reshape(NCHUNK, C))
    return _gather_ctx_call()(context, idx2d)
